# baseline (speedup 1.0000x reference)
import jax
import jax.numpy as jnp
from jax import lax
from jax.experimental import pallas as pl
from jax.experimental.pallas import tpu as pltpu
from jax.experimental.pallas import tpu_sc as plsc

VOCAB = 49408
B = 256
L = 77
D = 1024
NVEC = 8

NC = 2
NS = 16
NW = NC * NS
N = B * L
PER_W = N // NW
BPW = B // NW
CHUNK = 40
NBUF = 3
NFULL = PER_W // CHUNK
TAIL = PER_W - NFULL * CHUNK


def _gather_kernel(ids_hbm, table_hbm, out_hbm, ids_v, *bufs_and_sems):
    bufs = bufs_and_sems[:NBUF]
    gsems = bufs_and_sems[NBUF:2 * NBUF]
    wsems = bufs_and_sems[2 * NBUF:3 * NBUF]
    wid = lax.axis_index("s") * NC + lax.axis_index("c")
    base = wid * PER_W

    pltpu.sync_copy(ids_hbm.at[pl.ds(base, PER_W)], ids_v)

    nchunks = NFULL + 1
    sizes = [CHUNK] * NFULL + [TAIL]

    def gather(c):
        return pltpu.async_copy(
            table_hbm.at[ids_v.at[pl.ds(c * CHUNK, sizes[c])]],
            bufs[c % NBUF].at[pl.ds(0, sizes[c])], gsems[c % NBUF])

    def writeback(c):
        return pltpu.async_copy(
            bufs[c % NBUF].at[pl.ds(0, sizes[c])],
            out_hbm.at[pl.ds(base + c * CHUNK, sizes[c])], wsems[c % NBUF])

    g = [None] * nchunks
    w = [None] * nchunks
    for c in range(nchunks):
        if c >= NBUF:
            w[c - NBUF].wait()
        g[c] = gather(c)
        if c >= 1:
            g[c - 1].wait()
            w[c - 1] = writeback(c - 1)
    g[nchunks - 1].wait()
    w[nchunks - 1] = writeback(nchunks - 1)
    for c in range(max(0, nchunks - NBUF), nchunks - 1):
        w[c].wait()
    w[nchunks - 1].wait()


def _ti_kernel(ti_hbm, dest_hbm, out_ref, ti_v, didx_v, dsem):
    wid = lax.axis_index("s") * NC + lax.axis_index("c")
    pltpu.sync_copy(ti_hbm, ti_v.at[pl.ds(0, NVEC)])
    pltpu.sync_copy(ti_hbm, ti_v.at[pl.ds(NVEC, NVEC)])
    for g in range(BPW // 2):
        pltpu.sync_copy(dest_hbm.at[wid, g], didx_v)
        pltpu.async_copy(ti_v, out_ref.at[didx_v], dsem).wait()


@jax.jit
def kernel(input_ids, table, ti_emb, offsets):
    ids_lm = input_ids.T.reshape(N)
    lane = jnp.arange(16, dtype=jnp.int32)
    grp = jnp.arange(BPW // 2, dtype=jnp.int32)
    b = (jnp.arange(NW, dtype=jnp.int32)[:, None, None] * BPW
         + grp[None, :, None] * 2 + (lane[None, None, :] >> 3))
    dest = (offsets[b] + 1 + (lane[None, None, :] & 7)) * B + b

    mesh = plsc.VectorSubcoreMesh(core_axis_name="c", subcore_axis_name="s")
    out2 = pl.kernel(
        _gather_kernel,
        out_type=jax.ShapeDtypeStruct((N, D), jnp.float32),
        mesh=mesh,
        scratch_types=(
            [pltpu.VMEM((PER_W,), jnp.int32)]
            + [pltpu.VMEM((CHUNK, D), jnp.float32)] * NBUF
            + [pltpu.SemaphoreType.DMA] * (2 * NBUF)
        ),
    )(ids_lm, table)

    out_ref = jax.new_ref(out2)
    pl.kernel(
        _ti_kernel,
        out_type=(),
        mesh=mesh,
        scratch_types=[
            pltpu.VMEM((16, D), jnp.float32),
            pltpu.VMEM((16,), jnp.int32),
            pltpu.SemaphoreType.DMA,
        ],
    )(ti_emb, dest, out_ref)
    out = jax.freeze(out_ref)
    return out.reshape(L, B, D).transpose(1, 0, 2)


# speedup vs baseline: 1.0470x; 1.0470x over previous
import jax
import jax.numpy as jnp
from jax import lax
from jax.experimental import pallas as pl
from jax.experimental.pallas import tpu as pltpu
from jax.experimental.pallas import tpu_sc as plsc

VOCAB = 49408
B = 256
L = 77
D = 1024
NVEC = 8

NC = 2
NS = 16
NW = NC * NS
N = B * L
PER_W = N // NW
BPW = B // NW
CHUNK = 56
NBUF = 2
NFULL = PER_W // CHUNK
TAIL = PER_W - NFULL * CHUNK


def _gather_kernel(ids_hbm, table_hbm, out_hbm, ids_v, *bufs_and_sems):
    bufs = bufs_and_sems[:NBUF]
    gsems = bufs_and_sems[NBUF:2 * NBUF]
    wsems = bufs_and_sems[2 * NBUF:3 * NBUF]
    wid = lax.axis_index("s") * NC + lax.axis_index("c")
    base = wid * PER_W

    pltpu.sync_copy(ids_hbm.at[pl.ds(base, PER_W)], ids_v)

    sizes = [CHUNK] * NFULL + ([TAIL] if TAIL else [])
    nchunks = len(sizes)

    def gather(c):
        return pltpu.async_copy(
            table_hbm.at[ids_v.at[pl.ds(c * CHUNK, sizes[c])]],
            bufs[c % NBUF].at[pl.ds(0, sizes[c])], gsems[c % NBUF])

    def writeback(c):
        return pltpu.async_copy(
            bufs[c % NBUF].at[pl.ds(0, sizes[c])],
            out_hbm.at[pl.ds(base + c * CHUNK, sizes[c])], wsems[c % NBUF])

    g = [None] * nchunks
    w = [None] * nchunks
    for c in range(nchunks):
        if c >= NBUF:
            w[c - NBUF].wait()
        g[c] = gather(c)
        if c >= 1:
            g[c - 1].wait()
            w[c - 1] = writeback(c - 1)
    g[nchunks - 1].wait()
    w[nchunks - 1] = writeback(nchunks - 1)
    for c in range(max(0, nchunks - NBUF), nchunks - 1):
        w[c].wait()
    w[nchunks - 1].wait()


def _ti_kernel(ti2_hbm, dest_hbm, out_ref, ti_v, didx_v, dsem):
    wid = lax.axis_index("s") * NC + lax.axis_index("c")
    pltpu.sync_copy(ti2_hbm, ti_v)
    for g in range(BPW // 2):
        pltpu.sync_copy(dest_hbm.at[wid, g], didx_v)
        pltpu.async_copy(ti_v, out_ref.at[didx_v], dsem).wait()


@jax.jit
def kernel(input_ids, table, ti_emb, offsets):
    ids_lm = input_ids.T.reshape(N)
    ti2 = jnp.concatenate([ti_emb, ti_emb], axis=0)
    lane = jnp.arange(16, dtype=jnp.int32)
    grp = jnp.arange(BPW // 2, dtype=jnp.int32)
    b = (jnp.arange(NW, dtype=jnp.int32)[:, None, None] * BPW
         + grp[None, :, None] * 2 + (lane[None, None, :] >> 3))
    dest = (offsets[b] + 1 + (lane[None, None, :] & 7)) * B + b

    mesh = plsc.VectorSubcoreMesh(core_axis_name="c", subcore_axis_name="s")
    out2 = pl.kernel(
        _gather_kernel,
        out_type=jax.ShapeDtypeStruct((N, D), jnp.float32),
        mesh=mesh,
        scratch_types=(
            [pltpu.VMEM((PER_W,), jnp.int32)]
            + [pltpu.VMEM((CHUNK, D), jnp.float32)] * NBUF
            + [pltpu.SemaphoreType.DMA] * (2 * NBUF)
        ),
    )(ids_lm, table)

    out_ref = jax.new_ref(out2)
    pl.kernel(
        _ti_kernel,
        out_type=(),
        mesh=mesh,
        scratch_types=[
            pltpu.VMEM((16, D), jnp.float32),
            pltpu.VMEM((16,), jnp.int32),
            pltpu.SemaphoreType.DMA,
        ],
    )(ti2, dest, out_ref)
    out = jax.freeze(out_ref)
    return out.reshape(L, B, D).transpose(1, 0, 2)


# speedup vs baseline: 1.0637x; 1.0160x over previous
import jax
import jax.numpy as jnp
from jax import lax
from jax.experimental import pallas as pl
from jax.experimental.pallas import tpu as pltpu
from jax.experimental.pallas import tpu_sc as plsc

VOCAB = 49408
B = 256
L = 77
D = 1024
NVEC = 8

NC = 2
NS = 16
NW = NC * NS
N = B * L
PER_W = N // NW
BPW = B // NW
CHUNK = 56
NBUF = 2
NFULL = PER_W // CHUNK
TAIL = PER_W - NFULL * CHUNK


def _gather_kernel(ids_hbm, table_hbm, out_hbm, ids_v, *bufs_and_sems):
    bufs = bufs_and_sems[:NBUF]
    gsems = bufs_and_sems[NBUF:2 * NBUF]
    wsems = bufs_and_sems[2 * NBUF:3 * NBUF]
    wid = lax.axis_index("s") * NC + lax.axis_index("c")
    base = wid * PER_W

    pltpu.sync_copy(ids_hbm.at[pl.ds(base, PER_W)], ids_v)

    sizes = [CHUNK] * NFULL + ([TAIL] if TAIL else [])
    nchunks = len(sizes)

    def gather(c):
        return pltpu.async_copy(
            table_hbm.at[ids_v.at[pl.ds(c * CHUNK, sizes[c])]],
            bufs[c % NBUF].at[pl.ds(0, sizes[c])], gsems[c % NBUF])

    def writeback(c):
        return pltpu.async_copy(
            bufs[c % NBUF].at[pl.ds(0, sizes[c])],
            out_hbm.at[pl.ds(base + c * CHUNK, sizes[c])], wsems[c % NBUF])

    g = [None] * nchunks
    w = [None] * nchunks
    for c in range(nchunks):
        if c >= NBUF:
            w[c - NBUF].wait()
        g[c] = gather(c)
        if c >= 1:
            g[c - 1].wait()
            w[c - 1] = writeback(c - 1)
    g[nchunks - 1].wait()
    w[nchunks - 1] = writeback(nchunks - 1)
    for c in range(max(0, nchunks - NBUF), nchunks - 1):
        w[c].wait()
    w[nchunks - 1].wait()


def _ti_kernel(ti2_hbm, dest_hbm, out_ref,
               ti_v, didx0, didx1, didx2, didx3, isem, dsem):
    wid = lax.axis_index("s") * NC + lax.axis_index("c")
    didxs = (didx0, didx1, didx2, didx3)
    ic = [pltpu.async_copy(dest_hbm.at[wid, g], didxs[g], isem)
          for g in range(BPW // 2)]
    pltpu.sync_copy(ti2_hbm, ti_v)
    scat = []
    for g in range(BPW // 2):
        ic[g].wait()
        scat.append(pltpu.async_copy(ti_v, out_ref.at[didxs[g]], dsem))
    for s in scat:
        s.wait()


@jax.jit
def kernel(input_ids, table, ti_emb, offsets):
    ids_lm = input_ids.T.reshape(N)
    ti2 = jnp.concatenate([ti_emb, ti_emb], axis=0)
    lane = jnp.arange(16, dtype=jnp.int32)
    grp = jnp.arange(BPW // 2, dtype=jnp.int32)
    b = (jnp.arange(NW, dtype=jnp.int32)[:, None, None] * BPW
         + grp[None, :, None] * 2 + (lane[None, None, :] >> 3))
    dest = (offsets[b] + 1 + (lane[None, None, :] & 7)) * B + b

    mesh = plsc.VectorSubcoreMesh(core_axis_name="c", subcore_axis_name="s")
    out2 = pl.kernel(
        _gather_kernel,
        out_type=jax.ShapeDtypeStruct((N, D), jnp.float32),
        mesh=mesh,
        scratch_types=(
            [pltpu.VMEM((PER_W,), jnp.int32)]
            + [pltpu.VMEM((CHUNK, D), jnp.float32)] * NBUF
            + [pltpu.SemaphoreType.DMA] * (2 * NBUF)
        ),
    )(ids_lm, table)

    out_ref = jax.new_ref(out2)
    pl.kernel(
        _ti_kernel,
        out_type=(),
        mesh=mesh,
        scratch_types=(
            [pltpu.VMEM((16, D), jnp.float32)]
            + [pltpu.VMEM((16,), jnp.int32)] * (BPW // 2)
            + [pltpu.SemaphoreType.DMA] * 2
        ),
    )(ti2, dest, out_ref)
    out = jax.freeze(out_ref)
    return out.reshape(L, B, D).transpose(1, 0, 2)
